# baseline (speedup 1.0000x reference)
import jax
import jax.numpy as jnp
from jax.experimental import pallas as pl
from jax.experimental.pallas import tpu as pltpu
from jax.experimental.pallas import tpu_sc as plsc

_N = 1024
_D = 64
_BM = 4096
_GW = 128


def _vq_block(e_ref, w_ref, idx_ref, w2_ref):
    e = e_ref[...]
    w = w_ref[...]
    dot = jax.lax.dot_general(e, w, (((1,), (1,)), ((), ())),
                              preferred_element_type=jnp.float32)
    e_sq = jnp.sum(e * e, axis=1, keepdims=True)
    w_sq = jnp.sum(w * w, axis=1)[None, :]
    dist = e_sq + w_sq - 2.0 * dot
    m = jnp.min(dist, axis=1, keepdims=True)
    iota = jax.lax.broadcasted_iota(jnp.int32, dist.shape, 1)
    idx = jnp.min(jnp.where(dist == m, iota, _N), axis=1)
    idx_ref[...] = idx[None, :]

    @pl.when(pl.program_id(0) == 0)
    def _():
        w2_ref[...] = jnp.concatenate([w, w], axis=1)


def _sc_gather(w2, idx_row, M):
    vector_mesh = plsc.VectorSubcoreMesh(
        core_axis_name="core", subcore_axis_name="subcore")

    @pl.kernel(out_type=jax.ShapeDtypeStruct((M, 2 * _D), jnp.float32),
               mesh=vector_mesh)
    def gather_kernel(w_hbm, i_hbm, o_hbm):
        def body(i_vmem, o_vmem):
            pltpu.sync_copy(w_hbm.at[i_vmem.at[0]], o_vmem)

        pltpu.emit_pipeline(
            body,
            grid=(M // _GW,),
            in_specs=[pl.BlockSpec((1, _GW), index_map=lambda i: (0, i))],
            out_specs=[pl.BlockSpec((_GW, 2 * _D), index_map=lambda i: (i, 0))],
            core_axis_name="subcore",
            dimension_semantics=(pltpu.PARALLEL,),
        )(i_hbm, o_hbm)

    return gather_kernel(w2, idx_row)


def kernel(x, W):
    perm = (0,) + tuple(range(2, x.ndim)) + (1,)
    encoded_permuted = jnp.transpose(x, perm)
    permuted_shape = encoded_permuted.shape
    encoded_flat = encoded_permuted.reshape(-1, permuted_shape[-1])
    M = encoded_flat.shape[0]

    idx_row, w2 = pl.pallas_call(
        _vq_block,
        grid=(M // _BM,),
        in_specs=[
            pl.BlockSpec((_BM, _D), lambda i: (i, 0)),
            pl.BlockSpec((_N, _D), lambda i: (0, 0)),
        ],
        out_specs=[
            pl.BlockSpec((1, _BM), lambda i: (0, i)),
            pl.BlockSpec((_N, 2 * _D), lambda i: (0, 0)),
        ],
        out_shape=[
            jax.ShapeDtypeStruct((1, M), jnp.int32),
            jax.ShapeDtypeStruct((_N, 2 * _D), jnp.float32),
        ],
        compiler_params=pltpu.CompilerParams(
            dimension_semantics=("arbitrary",),
        ),
    )(encoded_flat, W)

    q_wide = _sc_gather(w2, idx_row, M)
    quantized_flat = q_wide[:, :_D]

    codebook_indices = idx_row.reshape(M)
    num_dims = len(permuted_shape)
    quantized_permuted = quantized_flat.reshape(permuted_shape)
    old_dims = (0,) + (num_dims - 1,) + tuple(range(1, num_dims - 1))
    quantized = jnp.transpose(quantized_permuted, old_dims)
    return (encoded_flat, quantized_flat, codebook_indices, quantized)


# speedup vs baseline: 1.1236x; 1.1236x over previous
import jax
import jax.numpy as jnp
from jax.experimental import pallas as pl
from jax.experimental.pallas import tpu as pltpu
from jax.experimental.pallas import tpu_sc as plsc

_N = 1024
_D = 64
_BM = 4096
_GW = 128


def _vq_block(e_ref, w_ref, idx_ref, w2_ref):
    e = e_ref[...]
    w = w_ref[...]
    dot = jax.lax.dot_general(e, w, (((1,), (1,)), ((), ())),
                              preferred_element_type=jnp.float32)
    e_sq = jnp.sum(e * e, axis=1, keepdims=True)
    w_sq = jnp.sum(w * w, axis=1)[None, :]
    dist = e_sq + w_sq - 2.0 * dot
    m = jnp.min(dist, axis=1, keepdims=True)
    iota = jax.lax.broadcasted_iota(jnp.int32, dist.shape, 1)
    idx = jnp.min(jnp.where(dist == m, iota, _N), axis=1)
    idx_ref[...] = idx[None, :]

    @pl.when(pl.program_id(0) == 0)
    def _():
        w2_ref[...] = jnp.concatenate([w, w], axis=1)


def _sc_gather(w2, idx_row, M):
    vector_mesh = plsc.VectorSubcoreMesh(
        core_axis_name="core", subcore_axis_name="subcore")

    @pl.kernel(out_type=jax.ShapeDtypeStruct((M, 2 * _D), jnp.float32),
               mesh=vector_mesh)
    def gather_kernel(w_hbm, i_hbm, o_hbm):
        def body(i_vmem, o_vmem):
            pltpu.sync_copy(w_hbm.at[i_vmem.at[0]], o_vmem)

        pltpu.emit_pipeline(
            body,
            grid=(M // _GW,),
            in_specs=[pl.BlockSpec((1, _GW), index_map=lambda i: (0, i))],
            out_specs=[pl.BlockSpec((_GW, 2 * _D), index_map=lambda i: (i, 0))],
            core_axis_name=("core", "subcore"),
            dimension_semantics=(pltpu.PARALLEL,),
        )(i_hbm, o_hbm)

    return gather_kernel(w2, idx_row)


def kernel(x, W):
    perm = (0,) + tuple(range(2, x.ndim)) + (1,)
    encoded_permuted = jnp.transpose(x, perm)
    permuted_shape = encoded_permuted.shape
    encoded_flat = encoded_permuted.reshape(-1, permuted_shape[-1])
    M = encoded_flat.shape[0]

    idx_row, w2 = pl.pallas_call(
        _vq_block,
        grid=(M // _BM,),
        in_specs=[
            pl.BlockSpec((_BM, _D), lambda i: (i, 0)),
            pl.BlockSpec((_N, _D), lambda i: (0, 0)),
        ],
        out_specs=[
            pl.BlockSpec((1, _BM), lambda i: (0, i)),
            pl.BlockSpec((_N, 2 * _D), lambda i: (0, 0)),
        ],
        out_shape=[
            jax.ShapeDtypeStruct((1, M), jnp.int32),
            jax.ShapeDtypeStruct((_N, 2 * _D), jnp.float32),
        ],
        compiler_params=pltpu.CompilerParams(
            dimension_semantics=("arbitrary",),
        ),
    )(encoded_flat, W)

    q_wide = _sc_gather(w2, idx_row, M)
    quantized_flat = q_wide[:, :_D]

    codebook_indices = idx_row.reshape(M)
    num_dims = len(permuted_shape)
    quantized_permuted = quantized_flat.reshape(permuted_shape)
    old_dims = (0,) + (num_dims - 1,) + tuple(range(1, num_dims - 1))
    quantized = jnp.transpose(quantized_permuted, old_dims)
    return (encoded_flat, quantized_flat, codebook_indices, quantized)


# speedup vs baseline: 1.4513x; 1.2916x over previous
import jax
import jax.numpy as jnp
from jax.experimental import pallas as pl
from jax.experimental.pallas import tpu as pltpu

_N = 1024
_D = 64
_BM = 4096


def _vq_block(e_ref, w_ref, idx_ref, q_ref):
    e = e_ref[...]
    w = w_ref[...]
    dot_m2 = jax.lax.dot_general(e * -2.0, w, (((1,), (1,)), ((), ())),
                                 preferred_element_type=jnp.float32)
    e_sq = jnp.sum(e * e, axis=1, keepdims=True)
    w_sq = jnp.sum(w * w, axis=1)[None, :]
    dist = (e_sq + w_sq) + dot_m2
    m = jnp.min(dist, axis=1, keepdims=True)
    iota = jax.lax.broadcasted_iota(jnp.int32, dist.shape, 1)
    idx = jnp.min(jnp.where(dist == m, iota, _N), axis=1)
    idx_ref[...] = idx[:, None]
    onehot = (iota == idx[:, None]).astype(jnp.float32)
    q = jax.lax.dot_general(onehot, w, (((1,), (0,)), ((), ())),
                            preferred_element_type=jnp.float32)
    q_ref[...] = e + (q - e)


def kernel(x, W):
    perm = (0,) + tuple(range(2, x.ndim)) + (1,)
    encoded_permuted = jnp.transpose(x, perm)
    permuted_shape = encoded_permuted.shape
    encoded_flat = encoded_permuted.reshape(-1, permuted_shape[-1])
    M = encoded_flat.shape[0]

    idx2, q = pl.pallas_call(
        _vq_block,
        grid=(M // _BM,),
        in_specs=[
            pl.BlockSpec((_BM, _D), lambda i: (i, 0)),
            pl.BlockSpec((_N, _D), lambda i: (0, 0)),
        ],
        out_specs=[
            pl.BlockSpec((_BM, 1), lambda i: (i, 0)),
            pl.BlockSpec((_BM, _D), lambda i: (i, 0)),
        ],
        out_shape=[
            jax.ShapeDtypeStruct((M, 1), jnp.int32),
            jax.ShapeDtypeStruct((M, _D), jnp.float32),
        ],
        compiler_params=pltpu.CompilerParams(
            dimension_semantics=("arbitrary",),
        ),
    )(encoded_flat, W)

    codebook_indices = idx2.reshape(M)
    quantized_flat = q
    num_dims = len(permuted_shape)
    quantized_permuted = quantized_flat.reshape(permuted_shape)
    old_dims = (0,) + (num_dims - 1,) + tuple(range(1, num_dims - 1))
    quantized = jnp.transpose(quantized_permuted, old_dims)
    return (encoded_flat, quantized_flat, codebook_indices, quantized)


# speedup vs baseline: 1.4820x; 1.0212x over previous
import jax
import jax.numpy as jnp
from jax.experimental import pallas as pl
from jax.experimental.pallas import tpu as pltpu

_N = 1024
_D = 64
_BM = 4096


def _vq_block(e_ref, w_ref, idx_ref, q_ref):
    e = e_ref[...]
    w = w_ref[...]
    dot_m2 = jax.lax.dot_general(e * -2.0, w, (((1,), (1,)), ((), ())),
                                 preferred_element_type=jnp.float32)
    e_sq = jnp.sum(e * e, axis=1, keepdims=True)
    w_sq = jnp.sum(w * w, axis=1)[None, :]
    dist = (e_sq + w_sq) + dot_m2
    m = jnp.min(dist, axis=1, keepdims=True)
    iota = jax.lax.broadcasted_iota(jnp.int32, dist.shape, 1)
    idx = jnp.min(jnp.where(dist == m, iota, _N), axis=1)
    idx_ref[...] = idx[None, :]
    onehot = (iota == idx[:, None]).astype(jnp.float32)
    q = jax.lax.dot_general(onehot, w, (((1,), (0,)), ((), ())),
                            preferred_element_type=jnp.float32)
    q_ref[...] = e + (q - e)


def kernel(x, W):
    perm = (0,) + tuple(range(2, x.ndim)) + (1,)
    encoded_permuted = jnp.transpose(x, perm)
    permuted_shape = encoded_permuted.shape
    encoded_flat = encoded_permuted.reshape(-1, permuted_shape[-1])
    M = encoded_flat.shape[0]

    idx2, q = pl.pallas_call(
        _vq_block,
        grid=(M // _BM,),
        in_specs=[
            pl.BlockSpec((_BM, _D), lambda i: (i, 0)),
            pl.BlockSpec((_N, _D), lambda i: (0, 0)),
        ],
        out_specs=[
            pl.BlockSpec((1, _BM), lambda i: (0, i)),
            pl.BlockSpec((_BM, _D), lambda i: (i, 0)),
        ],
        out_shape=[
            jax.ShapeDtypeStruct((1, M), jnp.int32),
            jax.ShapeDtypeStruct((M, _D), jnp.float32),
        ],
        compiler_params=pltpu.CompilerParams(
            dimension_semantics=("arbitrary",),
        ),
    )(encoded_flat, W)

    codebook_indices = idx2.reshape(M)
    quantized_flat = q
    num_dims = len(permuted_shape)
    quantized_permuted = quantized_flat.reshape(permuted_shape)
    old_dims = (0,) + (num_dims - 1,) + tuple(range(1, num_dims - 1))
    quantized = jnp.transpose(quantized_permuted, old_dims)
    return (encoded_flat, quantized_flat, codebook_indices, quantized)


# speedup vs baseline: 1.5029x; 1.0141x over previous
import jax
import jax.numpy as jnp
from jax.experimental import pallas as pl
from jax.experimental.pallas import tpu as pltpu

_N = 1024
_D = 64
_BM = 4096


def _vq_block(e_ref, w_ref, idx_ref, q_ref):
    e = e_ref[...]
    w = w_ref[...]
    dot = jax.lax.dot_general(e, w, (((1,), (1,)), ((), ())),
                              preferred_element_type=jnp.float32)
    e_sq = jnp.sum(e * e, axis=1, keepdims=True)
    w_sq = jnp.sum(w * w, axis=1)[None, :]
    dist = e_sq + w_sq - 2.0 * dot
    m = jnp.min(dist, axis=1, keepdims=True)
    iota = jax.lax.broadcasted_iota(jnp.int32, dist.shape, 1)
    idx = jnp.min(jnp.where(dist == m, iota, _N), axis=1)
    idx_ref[...] = idx[None, :]
    onehot = (iota == idx[:, None]).astype(jnp.float32)
    q = jax.lax.dot_general(onehot, w, (((1,), (0,)), ((), ())),
                            preferred_element_type=jnp.float32)
    q_ref[...] = e + (q - e)


def kernel(x, W):
    perm = (0,) + tuple(range(2, x.ndim)) + (1,)
    encoded_permuted = jnp.transpose(x, perm)
    permuted_shape = encoded_permuted.shape
    encoded_flat = encoded_permuted.reshape(-1, permuted_shape[-1])
    M = encoded_flat.shape[0]

    idx2, q = pl.pallas_call(
        _vq_block,
        grid=(M // _BM,),
        in_specs=[
            pl.BlockSpec((_BM, _D), lambda i: (i, 0)),
            pl.BlockSpec((_N, _D), lambda i: (0, 0)),
        ],
        out_specs=[
            pl.BlockSpec((1, _BM), lambda i: (0, i)),
            pl.BlockSpec((_BM, _D), lambda i: (i, 0)),
        ],
        out_shape=[
            jax.ShapeDtypeStruct((1, M), jnp.int32),
            jax.ShapeDtypeStruct((M, _D), jnp.float32),
        ],
        compiler_params=pltpu.CompilerParams(
            dimension_semantics=("arbitrary",),
        ),
    )(encoded_flat, W)

    codebook_indices = idx2.reshape(M)
    quantized_flat = q
    num_dims = len(permuted_shape)
    quantized_permuted = quantized_flat.reshape(permuted_shape)
    old_dims = (0,) + (num_dims - 1,) + tuple(range(1, num_dims - 1))
    quantized = jnp.transpose(quantized_permuted, old_dims)
    return (encoded_flat, quantized_flat, codebook_indices, quantized)
